# baseline (speedup 1.0000x reference)
import functools

import jax
import jax.numpy as jnp
from jax import lax
from jax.experimental import pallas as pl
from jax.experimental.pallas import tpu as pltpu
from jax.experimental.pallas import tpu_sc as plsc

_NC = 2
_NS = 16
_L = 16
_NB = 4
_BC = 8
_NR = 4


def _mm_body(x_ref, w_ref, o_ref):
    p = jnp.dot(x_ref[...], w_ref[...], preferred_element_type=jnp.float32)
    dh = o_ref.shape[2]
    o_ref[0] = p[:, :dh]
    o_ref[1] = p[:, dh:]


def _make_sc_scatter(N, D, NCH, CH):
    DH = D // _NC
    NBLK = NCH // _BC
    BE = _BC * CH
    assert NBLK * _BC == NCH and CH % _L == 0 and CH <= 128 and DH % _L == 0
    rps = (N // _NS) // 8 * 8
    tail = N - _NS * rps
    mesh = plsc.VectorSubcoreMesh(core_axis_name="c", subcore_axis_name="s")

    @functools.partial(
        pl.kernel,
        mesh=mesh,
        out_type=jax.ShapeDtypeStruct((N, D), jnp.float32),
        scratch_types=[
            pltpu.VMEM((_NR, BE), jnp.int32),
            pltpu.VMEM((_NR, _BC, CH), jnp.int32),
            pltpu.VMEM((_NR, BE), jnp.float32),
            pltpu.VMEM((_NB, CH, DH), jnp.float32),
            pltpu.VMEM_SHARED((N, DH), jnp.float32),
            pltpu.SemaphoreType.DMA,
            pltpu.SemaphoreType.DMA,
            pltpu.SemaphoreType.DMA,
        ],
        compiler_params=pltpu.CompilerParams(use_tc_tiling_on_sc=False),
    )
    def sc_kernel(support, col2, row4, w2, zeros, out,
                  col_v, row_v, w_v, rows_v, acc, gsem, ssem, tsem):
        c = lax.axis_index("c")
        s = lax.axis_index("s")
        pltpu.sync_copy(zeros.at[pl.ds(s * rps, rps)],
                        acc.at[pl.ds(s * rps, rps)])
        if tail:
            @pl.when(s == _NS - 1)
            def _():
                pltpu.sync_copy(zeros.at[pl.ds(_NS * rps, tail)],
                                acc.at[pl.ds(_NS * rps, tail)])

        def stage_start(j):
            r = j % _NR
            pltpu.async_copy(col2.at[s, pl.ds(j * BE, BE)], col_v.at[r], tsem)
            pltpu.async_copy(row4.at[s].at[pl.ds(j * _BC, _BC)],
                             row_v.at[r], tsem)
            pltpu.async_copy(w2.at[s, pl.ds(j * BE, BE)], w_v.at[r], tsem)

        def stage_wait(j):
            r = j % _NR
            pltpu.make_async_copy(col2.at[s, pl.ds(j * BE, BE)],
                                  col_v.at[r], tsem).wait()
            pltpu.make_async_copy(row4.at[s].at[pl.ds(j * _BC, _BC)],
                                  row_v.at[r], tsem).wait()
            pltpu.make_async_copy(w2.at[s, pl.ds(j * BE, BE)],
                                  w_v.at[r], tsem).wait()

        def gather_start(i, b):
            r = (i // _BC) % _NR
            q = i % _BC
            pltpu.async_copy(
                support.at[c].at[col_v.at[r, pl.ds(q * CH, CH)]],
                rows_v.at[b], gsem)

        def gather_wait(i, b):
            r = (i // _BC) % _NR
            q = i % _BC
            pltpu.make_async_copy(
                support.at[c].at[col_v.at[r, pl.ds(q * CH, CH)]],
                rows_v.at[b], gsem).wait()

        def scatter_start(i, b):
            r = (i // _BC) % _NR
            q = i % _BC
            pltpu.async_copy(rows_v.at[b], acc.at[row_v.at[r, q]],
                             ssem, add=True)

        def scatter_drain(i, b):
            r = (i // _BC) % _NR
            q = i % _BC
            pltpu.make_async_copy(rows_v.at[b], acc.at[row_v.at[r, q]],
                                  ssem).wait()

        def scale(i, b):
            r = (i // _BC) % _NR
            base = (i % _BC) * CH
            rb = rows_v.at[b]

            def edge_body(e, carry2):
                wg = w_v[r, pl.ds(base + e // _L * _L, _L)]
                wb = wg.at[jnp.full((_L,), e % _L, jnp.int32)].get(
                    mode="promise_in_bounds")
                for j in range(DH // _L):
                    sl = pl.ds(j * _L, _L)
                    rb[e, sl] = rb[e, sl] * wb
                return carry2

            lax.fori_loop(0, CH, edge_body, 0, unroll=4)

        def step(i, b):
            @pl.when(i >= _NB - 1)
            def _():
                scatter_drain(i - (_NB - 1), (i + 1) % _NB)

            @pl.when(i + 1 < NCH)
            def _():
                gather_start(i + 1, (i + 1) % _NB)

            gather_wait(i, b)
            scale(i, b)
            scatter_start(i, b)

        stage_start(0)
        stage_start(min(1, NBLK - 1))
        stage_wait(0)
        plsc.subcore_barrier()
        gather_start(0, 0)

        def block_body(j, carry):
            @pl.when(j + 2 < NBLK)
            def _():
                stage_start(j + 2)

            @pl.when(j + 1 < NBLK)
            def _():
                stage_wait(j + 1)

            for q in range(_BC):
                i = j * _BC + q
                step(i, i % _NB)
            return carry

        lax.fori_loop(0, NBLK, block_body, 0)
        for i in range(max(NCH - (_NB - 1), 0), NCH):
            scatter_drain(i, i % _NB)
        plsc.subcore_barrier()
        pltpu.sync_copy(acc.at[pl.ds(s * rps, rps)],
                        out.at[pl.ds(s * rps, rps), pl.ds(c * DH, DH)])
        if tail:
            @pl.when(s == _NS - 1)
            def _():
                pltpu.sync_copy(acc.at[pl.ds(_NS * rps, tail)],
                                out.at[pl.ds(_NS * rps, tail), pl.ds(c * DH, DH)])

    return sc_kernel


def kernel(x, edge_index, edge_weight, W):
    N, D_IN = x.shape
    D = W.shape[1]
    E = edge_weight.shape[0]
    DH = D // _NC
    CH = 128
    BE = _BC * CH
    NCH = -(-E // (_NS * BE)) * _BC
    EPS = NCH * CH
    pad = EPS * _NS - E

    blk = 1000
    support = pl.pallas_call(
        _mm_body,
        grid=(N // blk,),
        in_specs=[
            pl.BlockSpec((blk, D_IN), lambda i: (i, 0)),
            pl.BlockSpec((D_IN, D), lambda i: (0, 0)),
        ],
        out_specs=pl.BlockSpec((_NC, blk, DH), lambda i: (0, i, 0)),
        out_shape=jax.ShapeDtypeStruct((_NC, N, DH), jnp.float32),
    )(x, W)

    ipad = jnp.zeros((pad,), jnp.int32)
    row4 = jnp.concatenate([edge_index[0], ipad]).reshape(_NS, NCH, CH)
    col2 = jnp.concatenate([edge_index[1], ipad]).reshape(_NS, EPS)
    w2 = jnp.concatenate([edge_weight, jnp.zeros((pad,), jnp.float32)]
                         ).reshape(_NS, EPS)
    zeros = jnp.zeros((N, DH), jnp.float32)

    return _make_sc_scatter(N, D, NCH, CH)(support, col2, row4, w2, zeros)


# speedup vs baseline: 1.4425x; 1.4425x over previous
import functools

import jax
import jax.numpy as jnp
from jax import lax
from jax.experimental import pallas as pl
from jax.experimental.pallas import tpu as pltpu
from jax.experimental.pallas import tpu_sc as plsc

_NC = 2
_NS = 16
_L = 16
_NB = 3


def _mm_body(x_ref, w_ref, o_ref):
    p = jnp.dot(x_ref[...], w_ref[...], preferred_element_type=jnp.float32)
    dh = o_ref.shape[2]
    o_ref[0] = p[:, :dh]
    o_ref[1] = p[:, dh:]


def _make_sc_scatter(N, D, NCH, CH):
    DH = D // _NC
    EPS = NCH * CH
    assert CH % _L == 0 and CH <= 128 and DH % _L == 0
    rps = (N // _NS) // 8 * 8
    tail = N - _NS * rps
    mesh = plsc.VectorSubcoreMesh(core_axis_name="c", subcore_axis_name="s")

    @functools.partial(
        pl.kernel,
        mesh=mesh,
        out_type=jax.ShapeDtypeStruct((N, D), jnp.float32),
        scratch_types=[
            pltpu.VMEM((EPS,), jnp.int32),
            pltpu.VMEM((NCH, CH), jnp.int32),
            pltpu.VMEM((EPS,), jnp.float32),
            pltpu.VMEM((_NB, CH, DH), jnp.float32),
            pltpu.VMEM_SHARED((N, DH), jnp.float32),
            pltpu.SemaphoreType.DMA,
            pltpu.SemaphoreType.DMA,
        ],
        compiler_params=pltpu.CompilerParams(use_tc_tiling_on_sc=False),
    )
    def sc_kernel(support, col2, row3, w2, zeros, out,
                  col_v, row_v, w_v, rows_v, acc, gsem, ssem):
        c = lax.axis_index("c")
        s = lax.axis_index("s")
        pltpu.sync_copy(zeros.at[pl.ds(s * rps, rps)],
                        acc.at[pl.ds(s * rps, rps)])
        if tail:
            @pl.when(s == _NS - 1)
            def _():
                pltpu.sync_copy(zeros.at[pl.ds(_NS * rps, tail)],
                                acc.at[pl.ds(_NS * rps, tail)])
        pltpu.sync_copy(col2.at[s], col_v)
        pltpu.sync_copy(row3.at[s], row_v)
        pltpu.sync_copy(w2.at[s], w_v)
        plsc.subcore_barrier()

        def gather_start(i, b):
            pltpu.async_copy(support.at[c].at[col_v.at[pl.ds(i * CH, CH)]],
                             rows_v.at[b], gsem)

        def gather_wait(i, b):
            pltpu.make_async_copy(
                support.at[c].at[col_v.at[pl.ds(i * CH, CH)]],
                rows_v.at[b], gsem).wait()

        def scatter_start(i, b):
            pltpu.async_copy(rows_v.at[b], acc.at[row_v.at[i]], ssem, add=True)

        def scatter_wait(i, b):
            pltpu.make_async_copy(rows_v.at[b], acc.at[row_v.at[i]], ssem).wait()

        def scale(i, b):
            rb = rows_v.at[b]

            def edge_body(e, carry2):
                wg = w_v[pl.ds(i * CH + e // _L * _L, _L)]
                wb = wg.at[jnp.full((_L,), e % _L, jnp.int32)].get(
                    mode="promise_in_bounds")
                for j in range(DH // _L):
                    sl = pl.ds(j * _L, _L)
                    rb[e, sl] = rb[e, sl] * wb
                return carry2

            lax.fori_loop(0, CH, edge_body, 0, unroll=8)

        def step(i, b):
            @pl.when(i >= _NB - 1)
            def _():
                scatter_wait(i - (_NB - 1), (i + 1) % _NB)

            @pl.when(i + 1 < NCH)
            def _():
                gather_start(i + 1, (i + 1) % _NB)

            gather_wait(i, b)
            scale(i, b)
            scatter_start(i, b)

        gather_start(0, 0)

        def ring_body(p, carry):
            for b in range(_NB):
                step(_NB * p + b, b)
            return carry

        lax.fori_loop(0, NCH // _NB, ring_body, 0)
        for i in range(NCH // _NB * _NB, NCH):
            step(i, i % _NB)
        for i in range(max(NCH - (_NB - 1), 0), NCH):
            scatter_wait(i, i % _NB)
        plsc.subcore_barrier()
        pltpu.sync_copy(acc.at[pl.ds(s * rps, rps)],
                        out.at[pl.ds(s * rps, rps), pl.ds(c * DH, DH)])
        if tail:
            @pl.when(s == _NS - 1)
            def _():
                pltpu.sync_copy(acc.at[pl.ds(_NS * rps, tail)],
                                out.at[pl.ds(_NS * rps, tail), pl.ds(c * DH, DH)])

    return sc_kernel


def kernel(x, edge_index, edge_weight, W):
    N, D_IN = x.shape
    D = W.shape[1]
    E = edge_weight.shape[0]
    DH = D // _NC
    CH = 128
    NCH = -(-E // (_NS * CH))
    EPS = NCH * CH
    pad = EPS * _NS - E

    blk = 1000
    support = pl.pallas_call(
        _mm_body,
        grid=(N // blk,),
        in_specs=[
            pl.BlockSpec((blk, D_IN), lambda i: (i, 0)),
            pl.BlockSpec((D_IN, D), lambda i: (0, 0)),
        ],
        out_specs=pl.BlockSpec((_NC, blk, DH), lambda i: (0, i, 0)),
        out_shape=jax.ShapeDtypeStruct((_NC, N, DH), jnp.float32),
    )(x, W)

    ipad = jnp.zeros((pad,), jnp.int32)
    row3 = jnp.concatenate([edge_index[0], ipad]).reshape(_NS, NCH, CH)
    col2 = jnp.concatenate([edge_index[1], ipad]).reshape(_NS, EPS)
    w2 = jnp.concatenate([edge_weight, jnp.zeros((pad,), jnp.float32)]
                         ).reshape(_NS, EPS)
    zeros = jnp.zeros((N, DH), jnp.float32)

    return _make_sc_scatter(N, D, NCH, CH)(support, col2, row3, w2, zeros)
